# baseline (speedup 1.0000x reference)
import functools

import jax
import jax.numpy as jnp
from jax import lax
from jax.experimental import pallas as pl
from jax.experimental.pallas import tpu as pltpu
from jax.experimental.pallas import tpu_sc as plsc

EMBED = 64
SB = 128
NE = 4
NR = 3
UNROLL = 12
Q = 25000
ACC_ROWS = 25088
TAIL_PAD = 1024


def _mesh():
    return plsc.VectorSubcoreMesh(core_axis_name="c", subcore_axis_name="s")


def _make_layer(PA, U, N):
    share = PA // 16
    nchunk = share // SB
    assert nchunk % UNROLL == 0
    nouter = nchunk // UNROLL

    @functools.partial(
        pl.kernel,
        mesh=_mesh(),
        compiler_params=pltpu.CompilerParams(use_tc_tiling_on_sc=False),
        out_type=jax.ShapeDtypeStruct((N, EMBED), jnp.float32),
        scratch_types=[
            [pltpu.VMEM((2, SB), jnp.int32) for _ in range(NE)],
            [pltpu.VMEM((SB,), jnp.float32) for _ in range(NE)],
            [pltpu.VMEM((1, SB), jnp.int32) for _ in range(NE)],
            [pltpu.VMEM((SB, EMBED), jnp.float32) for _ in range(NR)],
            pltpu.VMEM_SHARED((ACC_ROWS, EMBED), jnp.float32),
            pltpu.SemaphoreType.DMA,
            pltpu.SemaphoreType.DMA,
            pltpu.SemaphoreType.DMA,
        ],
    )
    def layer(f_hbm, rc_hbm, val_hbm, out_hbm,
              rcv, valv, idxd, rows, acc, sem_e, sem_g, sem_s):
        c = lax.axis_index("c")
        s = lax.axis_index("s")

        def fire_e(ch, ph):
            e0 = c * PA + s * share + ch * SB
            pltpu.async_copy(rc_hbm.at[:, pl.ds(e0, SB)], rcv[ph % NE], sem_e)
            pltpu.async_copy(val_hbm.at[pl.ds(e0, SB)], valv[ph % NE], sem_e)

        def drain_e(ch, ph):
            e0 = c * PA + s * share + ch * SB
            pltpu.make_async_copy(rc_hbm.at[:, pl.ds(e0, SB)],
                                  rcv[ph % NE], sem_e).wait()
            pltpu.make_async_copy(val_hbm.at[pl.ds(e0, SB)],
                                  valv[ph % NE], sem_e).wait()

        def transform(ph, base):
            rv = rcv[ph % NE]
            dv = idxd[ph % NE]
            for i in range(SB // 16):
                r16 = rv[0, pl.ds(i * 16, 16)]
                idx = r16 - base
                ok = (idx >= 0) & (idx < Q)
                dv[0, pl.ds(i * 16, 16)] = jnp.where(ok, idx, Q)

        def fire_g(ph):
            pltpu.async_copy(f_hbm.at[rcv[ph % NE].at[1]],
                             rows[ph % NR], sem_g)

        def drain_g(ph):
            pltpu.make_async_copy(f_hbm.at[rcv[ph % NE].at[1]],
                                  rows[ph % NR], sem_g).wait()

        def scale(ph):
            rb = rows[ph % NR]
            vb = valv[ph % NE]

            def sbody(j, _):
                vv = vb[pl.ds(j * 16, 16)]
                for e in range(16):
                    v = vv[e]
                    for t in range(EMBED // 16):
                        sl = pl.ds(t * 16, 16)
                        rb[j * 16 + e, sl] = rb[j * 16 + e, sl] * v
                return 0
            lax.fori_loop(0, SB // 16, sbody, 0)

        def fire_w(ph):
            pltpu.async_copy(rows[ph % NR], acc.at[idxd[ph % NE].at[0]],
                             sem_s, add=True)

        def drain_w(ph):
            pltpu.make_async_copy(rows[ph % NR], acc.at[idxd[ph % NE].at[0]],
                                  sem_s).wait()

        def zvbody(j, _):
            for t in range(EMBED // 16):
                rows[0][j, pl.ds(t * 16, 16)] = jnp.zeros((16,), jnp.float32)
            return 0

        for p in range(2):
            base = c * U + p * Q

            lax.fori_loop(0, SB, zvbody, 0)

            def zbody(kk, _):
                r = (s + 16 * kk) * 128

                @pl.when(r < ACC_ROWS)
                def _():
                    pltpu.sync_copy(rows[0], acc.at[pl.ds(r, 128)])
                return 0
            lax.fori_loop(0, 13, zbody, 0)
            plsc.subcore_barrier()

            fire_e(0, 0)
            fire_e(1, 1)
            fire_e(2, 2)
            drain_e(0, 0)
            transform(0, base)
            fire_g(0)
            drain_e(1, 1)
            transform(1, base)
            fire_g(1)

            def outer(i, _):
                c0 = i * UNROLL
                for u in range(UNROLL):
                    ch = c0 + u
                    if u == 0:
                        @pl.when(i > 0)
                        def _():
                            drain_w(UNROLL - 1)
                    else:
                        drain_w(u - 1)
                    drain_e(ch + 2, u + 2)
                    transform(u + 2, base)
                    fire_g(u + 2)
                    fire_e(ch + 3, u + 3)
                    drain_g(u)
                    scale(u)
                    fire_w(u)
                return 0
            lax.fori_loop(0, nouter, outer, 0)

            drain_e(nchunk + 2, 2)
            drain_g(0)
            drain_g(1)
            drain_w(UNROLL - 1)
            plsc.subcore_barrier()

            def dbody(kk, _):
                r = (s + 16 * kk) * 128

                @pl.when(r <= 24832)
                def _():
                    pltpu.sync_copy(acc.at[pl.ds(r, 128)], rows[0])
                    pltpu.sync_copy(rows[0], out_hbm.at[pl.ds(base + r, 128)])
                return 0
            lax.fori_loop(0, 13, dbody, 0)

            @pl.when(s == 3)
            def _():
                pltpu.sync_copy(acc.at[pl.ds(24960, 40)],
                                rows[1].at[pl.ds(0, 40)])
                pltpu.sync_copy(rows[1].at[pl.ds(0, 40)],
                                out_hbm.at[pl.ds(base + 24960, 40)])
            plsc.subcore_barrier()

    return layer


def _make_final(B, N):
    shb = B // 32
    nb = shb // 128

    @functools.partial(
        pl.kernel,
        mesh=_mesh(),
        compiler_params=pltpu.CompilerParams(use_tc_tiling_on_sc=False),
        out_type=[jax.ShapeDtypeStruct((B, EMBED), jnp.float32),
                  jax.ShapeDtypeStruct((B, EMBED), jnp.float32)],
        scratch_types=[
            pltpu.VMEM((shb,), jnp.int32),
            pltpu.VMEM((shb,), jnp.int32),
            pltpu.VMEM((nb, 128), jnp.int32),
            pltpu.VMEM((nb, 128), jnp.int32),
            pltpu.VMEM((128, EMBED), jnp.float32),
            pltpu.VMEM((128, EMBED), jnp.float32),
            pltpu.VMEM((128, EMBED), jnp.float32),
            pltpu.SemaphoreType.DMA,
        ],
    )
    def final(f0, f1, f2, f3, uidx, iidx, uout, iout,
              uv1, iv1, uv2, iv2, gbuf, usum, isum, sem):
        c = lax.axis_index("c")
        s = lax.axis_index("s")
        wid = c * 16 + s
        b0 = wid * shb
        pltpu.sync_copy(uidx.at[pl.ds(b0, shb)], uv1)
        pltpu.sync_copy(iidx.at[pl.ds(b0, shb)], iv1)
        for g in range(nb):
            for i in range(8):
                off = g * 128 + i * 16
                uv2[g, pl.ds(i * 16, 16)] = uv1[pl.ds(off, 16)]
                iv2[g, pl.ds(i * 16, 16)] = iv1[pl.ds(off, 16)]
        snaps = [f0, f1, f2, f3]
        for g in range(nb):
            for dst, iv, out in ((usum, uv2, uout), (isum, iv2, iout)):
                for l, f in enumerate(snaps):
                    pltpu.async_copy(f.at[iv.at[g]], gbuf, sem).wait()

                    def abody(j, _, l=l, dst=dst):
                        for t in range(EMBED // 16):
                            sl = pl.ds(t * 16, 16)
                            g16 = gbuf[j, sl]
                            if l == 0:
                                dst[j, sl] = g16
                            else:
                                dst[j, sl] = dst[j, sl] + g16
                        return 0
                    lax.fori_loop(0, 128, abody, 0)
                pltpu.sync_copy(dst, out.at[pl.ds(b0 + g * 128, 128)])

    return final


def _dot_body(u_ref, i_ref, o_ref):
    o_ref[...] = jnp.sum(u_ref[...] * i_ref[...], axis=1) * 0.0625


def kernel(uEmbd, iEmbd, L_val, L_row, L_col, userIdx, itemIdx):
    U = uEmbd.shape[0]
    N = U + iEmbd.shape[0]
    E = L_val.shape[0]
    E2 = E // 2
    B = userIdx.shape[0]

    unit = 16 * UNROLL * SB
    PA = ((E2 + unit - 1) // unit) * unit
    pad = PA - E2

    def pad_half(x, fill, dtype, tail):
        fills = jnp.full((pad,), fill, dtype)
        tfills = jnp.full((pad + tail,), fill, dtype)
        return jnp.concatenate([x[:E2].astype(dtype), fills,
                                x[E2:].astype(dtype), tfills])

    rowE = pad_half(L_row, -1, jnp.int32, TAIL_PAD)
    colE = pad_half(L_col, 0, jnp.int32, TAIL_PAD)
    valE = pad_half(L_val, 0.0, jnp.float32, TAIL_PAD)
    rcE = jnp.stack([rowE, colE])

    f0 = jnp.concatenate([uEmbd, iEmbd], axis=0)
    layer = _make_layer(PA, U, N)
    f1 = layer(f0, rcE, valE)
    f2 = layer(f1, rcE, valE)
    f3 = layer(f2, rcE, valE)

    final = _make_final(B, N)
    usum, isum = final(f0, f1, f2, f3, userIdx.astype(jnp.int32),
                       (itemIdx + U).astype(jnp.int32))
    return pl.pallas_call(
        _dot_body,
        out_shape=jax.ShapeDtypeStruct((B,), jnp.float32),
    )(usum, isum)
